# baseline (speedup 1.0000x reference)
import jax
import jax.numpy as jnp
from jax.experimental import pallas as pl

N = 10000
E = 160000
D = 128
K = 512


def _matmul_body(x_ref, w_ref, o_ref):
    o_ref[...] = jnp.dot(x_ref[...], w_ref[...],
                         preferred_element_type=jnp.float32)


def _matmul(x, w):
    return pl.pallas_call(
        _matmul_body,
        out_shape=jax.ShapeDtypeStruct((x.shape[0], w.shape[1]), jnp.float32),
    )(x, w)


def kernel(x, edge_index, edge_weight, batch, W_lin, b_lin, W_att, b_att,
           le_W1, le_b1, le_W2, le_W3, le_b3):
    n = x.shape[0]
    loops = jnp.arange(n)
    src = jnp.concatenate([edge_index[0], loops])
    dst = jnp.concatenate([edge_index[1], loops])
    w = jnp.concatenate([edge_weight, jnp.ones((n,), dtype=x.dtype)])

    p = _matmul(x, W_att[D:].reshape(D, 1))[:, 0]

    M = jax.ops.segment_max(x[src], dst, num_segments=n)
    q = _matmul(M, W_lin) + b_lin
    q = _matmul(q, W_att[:D].reshape(D, 1))[:, 0] + b_att

    score = q[dst] + p[src]
    score = jnp.where(score > 0, score, 0.2 * score)

    m = jax.ops.segment_max(score, dst, num_segments=n)
    e = jnp.exp(score - m[dst])
    denom = jax.ops.segment_sum(e, dst, num_segments=n)
    score = e / (denom[dst] + 1e-16)

    x_new = jax.ops.segment_sum(x[src] * score[:, None], dst, num_segments=n)

    lw = jnp.concatenate([le_W1, le_W2, le_W3], axis=1)
    abz = _matmul(x_new, lw)
    a = abz[:, 0] + le_b1[0]
    b2 = abz[:, 1]
    z3 = abz[:, 2] + le_b3[0]
    deg = jax.ops.segment_sum(jnp.ones_like(w), dst, num_segments=n)
    agg = deg * a - jax.ops.segment_sum(b2[src], dst, num_segments=n)
    fitness = jax.nn.sigmoid(agg + z3)

    _, perm = jax.lax.top_k(fitness, K)
    x_out = x_new[perm] * fitness[perm][:, None]
    batch_out = batch[perm]

    inv_perm = jnp.full((n,), K, dtype=jnp.int32).at[perm].set(
        jnp.arange(K, dtype=jnp.int32))
    col_pos = inv_perm[dst]
    S = jnp.zeros((n, K + 1), dtype=x.dtype).at[src, col_pos].add(score)[:, :K]
    T = jax.ops.segment_sum(w[:, None] * S[dst], src, num_segments=n)
    A_new = jnp.zeros((K + 1, K), dtype=x.dtype).at[col_pos].add(
        score[:, None] * T[src])[:K]
    A_new = A_new * (1.0 - jnp.eye(K, dtype=x.dtype))
    return (x_out, A_new, batch_out, perm)
